# baseline (speedup 1.0000x reference)
import functools

import jax
import jax.numpy as jnp
from jax.experimental import pallas as pl
from jax.experimental.pallas import tpu as pltpu

_B, _S, _D, _E, _DFF, _TOPK = 1, 2048, 1024, 8, 2048, 2
_TS = 512
_NS = _S // _TS


def _route_kernel(x_ref, wg_ref, logits_ref, wts_ref, idx_ref):
    x = x_ref[...]
    r = jnp.dot(x, wg_ref[...], preferred_element_type=jnp.float32)
    logits = jnp.mean(r, axis=0, keepdims=True)
    logits_ref[...] = logits
    m = jnp.max(logits)
    ex = jnp.exp(logits - m)
    p = ex / jnp.sum(ex)
    i1 = jnp.argmax(p)
    w1 = jnp.max(p)
    iota = jax.lax.broadcasted_iota(jnp.int32, (1, _E), 1)
    p2 = jnp.where(iota == i1, -jnp.inf, p)
    i2 = jnp.argmax(p2)
    w2 = jnp.max(p2)
    wts_ref[...] = jnp.concatenate(
        [w1.reshape(1, 1), w2.reshape(1, 1)], axis=1)
    idx_ref[...] = jnp.concatenate(
        [i1.astype(jnp.int32).reshape(1, 1), i2.astype(jnp.int32).reshape(1, 1)],
        axis=1)


def _ffn_kernel(idx_sm, wts_sm, x_ref, w1a_ref, w1b_ref, b1a_ref, b1b_ref,
                w2a_ref, w2b_ref, b2a_ref, b2b_ref, out_ref):
    x = x_ref[...].astype(jnp.bfloat16)
    ha = jnp.dot(x, w1a_ref[0].astype(jnp.bfloat16),
                 preferred_element_type=jnp.float32)
    ha = jax.nn.gelu(ha + b1a_ref[0])
    hb = jnp.dot(x, w1b_ref[0].astype(jnp.bfloat16),
                 preferred_element_type=jnp.float32)
    hb = jax.nn.gelu(hb + b1b_ref[0])
    oa = jnp.dot(ha.astype(jnp.bfloat16), w2a_ref[0].astype(jnp.bfloat16),
                 preferred_element_type=jnp.float32)
    ob = jnp.dot(hb.astype(jnp.bfloat16), w2b_ref[0].astype(jnp.bfloat16),
                 preferred_element_type=jnp.float32)
    w0 = wts_sm[0]
    w1 = wts_sm[1]
    bias = w0 * b2a_ref[0] + w1 * b2b_ref[0]
    out_ref[...] = w0 * oa + w1 * ob + bias


@jax.jit
def kernel(hidden_states, W_gate, W1, b1, W2, b2):
    x2 = hidden_states.reshape(_S, _D)

    logits, wts, idx = pl.pallas_call(
        _route_kernel,
        out_shape=(
            jax.ShapeDtypeStruct((1, _E), jnp.float32),
            jax.ShapeDtypeStruct((1, _TOPK), jnp.float32),
            jax.ShapeDtypeStruct((1, _TOPK), jnp.int32),
        ),
    )(x2, W_gate)

    grid_spec = pltpu.PrefetchScalarGridSpec(
        num_scalar_prefetch=2,
        grid=(_NS,),
        in_specs=[
            pl.BlockSpec((_TS, _D), lambda si, idx_s, wts_s: (si, 0)),
            pl.BlockSpec((1, _D, _DFF), lambda si, idx_s, wts_s: (idx_s[0], 0, 0)),
            pl.BlockSpec((1, _D, _DFF), lambda si, idx_s, wts_s: (idx_s[1], 0, 0)),
            pl.BlockSpec((1, 1, _DFF), lambda si, idx_s, wts_s: (idx_s[0], 0, 0)),
            pl.BlockSpec((1, 1, _DFF), lambda si, idx_s, wts_s: (idx_s[1], 0, 0)),
            pl.BlockSpec((1, _DFF, _D), lambda si, idx_s, wts_s: (idx_s[0], 0, 0)),
            pl.BlockSpec((1, _DFF, _D), lambda si, idx_s, wts_s: (idx_s[1], 0, 0)),
            pl.BlockSpec((1, 1, _D), lambda si, idx_s, wts_s: (idx_s[0], 0, 0)),
            pl.BlockSpec((1, 1, _D), lambda si, idx_s, wts_s: (idx_s[1], 0, 0)),
        ],
        out_specs=pl.BlockSpec((_TS, _D), lambda si, idx_s, wts_s: (si, 0)),
    )
    b1r = b1.reshape(_E, 1, _DFF)
    b2r = b2.reshape(_E, 1, _D)
    out = pl.pallas_call(
        _ffn_kernel,
        grid_spec=grid_spec,
        out_shape=jax.ShapeDtypeStruct((_S, _D), jnp.float32),
        compiler_params=pltpu.CompilerParams(
            dimension_semantics=("arbitrary",)),
    )(idx.reshape(_TOPK), wts.reshape(_TOPK), x2, W1, W1, b1r, b1r,
      W2, W2, b2r, b2r)

    return (out.reshape(_B, _S, _D), logits)
